# baseline (speedup 1.0000x reference)
import functools

import jax
import jax.numpy as jnp
from jax import lax
from jax.experimental import pallas as pl
from jax.experimental.pallas import tpu as pltpu
from jax.experimental.pallas import tpu_sc as plsc

HP, WP, HEADS = 32, 32, 16
NB = 32
N = HP * WP
WIDE = 2048
HALF = HEADS // 2


def _sc_gather_body(idxv_hbm, table_hbm, w2_hbm,
                    rowidx_v, ival_v, ival2_v, rows_v, trans_v, sem, *, h0):
    a = lax.axis_index("s") * 2 + lax.axis_index("c")
    for ch in range(4):
        d = lax.iota(jnp.int32, 16) + 16 * ch
        lo = 32 * ((31 - d) * 32 + a)
        hi = 32 * a + (d - 31)
        rowidx_v[pl.ds(16 * ch, 16)] = jnp.where(d <= 31, lo, hi)
    pltpu.async_copy(idxv_hbm.at[rowidx_v], ival_v, sem).wait()

    for ch in range(16):
        for u in range(8):
            d, k = (ch * 128 + u * 16) // 32, (u * 16) % 32
            ival2_v[ch, pl.ds(u * 16, 16)] = ival_v[d, pl.ds(k, 16)]

    for ch in range(16):
        pltpu.make_async_copy(
            table_hbm.at[ival2_v.at[ch]],
            rows_v.at[pl.ds(ch * 128, 128)], sem).start()
    for ch in range(16):
        pltpu.make_async_copy(
            table_hbm.at[ival2_v.at[ch]],
            rows_v.at[pl.ds(ch * 128, 128)], sem).wait()

    lanes = lax.iota(jnp.int32, 16)
    for hh in range(HALF):
        hvec = jnp.full((16,), h0 + hh, jnp.int32)

        def tr(c8, carry, hvec=hvec, hh=hh):
            for u in range(8):
                c = c8 * 8 + u
                x = plsc.load_gather(rows_v, [c * 16 + lanes, hvec])
                trans_v[hh, pl.ds(pl.multiple_of(c * 16, 16), 16)] = x
            return carry
        lax.fori_loop(0, WIDE // 128, tr, 0)
        pltpu.make_async_copy(trans_v.at[hh], w2_hbm.at[hh, a], sem).start()
    for hh in range(HALF):
        pltpu.make_async_copy(trans_v.at[hh], w2_hbm.at[hh, a], sem).wait()


def _expand_body(w_ref, o_ref):
    y = w_ref[0]
    for k in range(NB):
        off = 32 * (31 - k)
        o_ref[0, k] = y[:, off:off + 1024]


def _expand_body2(prev_ref, w_ref, o_ref):
    del prev_ref
    _expand_body(w_ref, o_ref)


def _sc_gather_half(h0):
    return pl.kernel(
        functools.partial(_sc_gather_body, h0=h0),
        out_type=jax.ShapeDtypeStruct((HALF, 32, WIDE), jnp.float32),
        mesh=plsc.VectorSubcoreMesh(core_axis_name="c", subcore_axis_name="s"),
        compiler_params=pltpu.CompilerParams(
            needs_layout_passes=False, use_tc_tiling_on_sc=False),
        scratch_types=[
            pltpu.VMEM((64,), jnp.int32),
            pltpu.VMEM((64, 32), jnp.int32),
            pltpu.VMEM((16, 128), jnp.int32),
            pltpu.VMEM((WIDE, HEADS), jnp.float32),
            pltpu.VMEM((HALF, WIDE), jnp.float32),
            pltpu.SemaphoreType.DMA,
        ],
    )


def kernel(bias_table, index):
    idxv = index.reshape((N * N) // 32, 32)
    w2a = _sc_gather_half(0)(idxv, bias_table)
    w2b = _sc_gather_half(HALF)(idxv, bias_table)
    out_shape = jax.ShapeDtypeStruct((HEADS, NB, 32, N), jnp.float32)
    out4 = pl.pallas_call(
        _expand_body,
        grid=(HALF,),
        in_specs=[pl.BlockSpec((1, 32, WIDE), lambda h: (h, 0, 0))],
        out_specs=pl.BlockSpec((1, NB, 32, N), lambda h: (h, 0, 0, 0)),
        out_shape=out_shape,
        compiler_params=pltpu.CompilerParams(
            dimension_semantics=("parallel",)),
    )(w2a)
    out4 = pl.pallas_call(
        _expand_body2,
        grid=(HALF,),
        in_specs=[
            pl.BlockSpec((1, 1, 8, 128), lambda h: (0, 0, 0, 0)),
            pl.BlockSpec((1, 32, WIDE), lambda h: (h, 0, 0)),
        ],
        out_specs=pl.BlockSpec((1, NB, 32, N), lambda h: (h + HALF, 0, 0, 0)),
        out_shape=out_shape,
        input_output_aliases={0: 0},
        compiler_params=pltpu.CompilerParams(
            dimension_semantics=("arbitrary",)),
    )(out4, w2b)
    return out4.reshape(HEADS, N, N)


# speedup vs baseline: 1.1437x; 1.1437x over previous
import jax
import jax.numpy as jnp
from jax import lax
from jax.experimental import pallas as pl
from jax.experimental.pallas import tpu as pltpu
from jax.experimental.pallas import tpu_sc as plsc

HP, WP, HEADS = 32, 32, 16
NB = 32
N = HP * WP
WIDE = 2048


def _sc_gather_body(idxv_hbm, table_hbm, w2_hbm,
                    rowidx_v, ival_v, ival2_v, rows_v, trans_v, sem):
    a = lax.axis_index("s") * 2 + lax.axis_index("c")
    for ch in range(4):
        d = lax.iota(jnp.int32, 16) + 16 * ch
        lo = 32 * ((31 - d) * 32 + a)
        hi = 32 * a + (d - 31)
        rowidx_v[pl.ds(16 * ch, 16)] = jnp.where(d <= 31, lo, hi)
    pltpu.async_copy(idxv_hbm.at[rowidx_v], ival_v, sem).wait()

    for ch in range(16):
        for u in range(8):
            d, k = (ch * 128 + u * 16) // 32, (u * 16) % 32
            ival2_v[ch, pl.ds(u * 16, 16)] = ival_v[d, pl.ds(k, 16)]

    for ch in range(16):
        pltpu.make_async_copy(
            table_hbm.at[ival2_v.at[ch]],
            rows_v.at[pl.ds(ch * 128, 128)], sem).start()
    for ch in range(16):
        pltpu.make_async_copy(
            table_hbm.at[ival2_v.at[ch]],
            rows_v.at[pl.ds(ch * 128, 128)], sem).wait()

    lanes = lax.iota(jnp.int32, 16)
    for h in range(HEADS):
        hvec = jnp.full((16,), h, jnp.int32)

        def tr(c8, carry, hvec=hvec):
            for u in range(8):
                c = c8 * 8 + u
                x = plsc.load_gather(rows_v, [c * 16 + lanes, hvec])
                trans_v[h, pl.ds(pl.multiple_of(c * 16, 16), 16)] = x
            return carry
        lax.fori_loop(0, WIDE // 128, tr, 0)
        pltpu.make_async_copy(trans_v.at[h], w2_hbm.at[h, a], sem).start()
    for h in range(HEADS):
        pltpu.make_async_copy(trans_v.at[h], w2_hbm.at[h, a], sem).wait()


def _expand_body(w_ref, o_ref):
    y = w_ref[0]
    for k in range(NB):
        off = 32 * (31 - k)
        o_ref[0, k] = y[:, off:off + 1024]


def kernel(bias_table, index):
    idxv = index.reshape((N * N) // 32, 32)
    sc_gather = pl.kernel(
        _sc_gather_body,
        out_type=jax.ShapeDtypeStruct((HEADS, 32, WIDE), jnp.float32),
        mesh=plsc.VectorSubcoreMesh(core_axis_name="c", subcore_axis_name="s"),
        compiler_params=pltpu.CompilerParams(
            needs_layout_passes=False, use_tc_tiling_on_sc=False),
        scratch_types=[
            pltpu.VMEM((64,), jnp.int32),
            pltpu.VMEM((64, 32), jnp.int32),
            pltpu.VMEM((16, 128), jnp.int32),
            pltpu.VMEM((WIDE, HEADS), jnp.float32),
            pltpu.VMEM((HEADS, WIDE), jnp.float32),
            pltpu.SemaphoreType.DMA,
        ],
    )
    w2 = sc_gather(idxv, bias_table)
    out4 = pl.pallas_call(
        _expand_body,
        grid=(HEADS,),
        in_specs=[pl.BlockSpec((1, 32, WIDE), lambda h: (h, 0, 0))],
        out_specs=pl.BlockSpec((1, NB, 32, N), lambda h: (h, 0, 0, 0)),
        out_shape=jax.ShapeDtypeStruct((HEADS, NB, 32, N), jnp.float32),
        compiler_params=pltpu.CompilerParams(
            dimension_semantics=("parallel",)),
    )(w2)
    return out4.reshape(HEADS, N, N)


# speedup vs baseline: 1.1557x; 1.0105x over previous
import jax
import jax.numpy as jnp
from jax import lax
from jax.experimental import pallas as pl
from jax.experimental.pallas import tpu as pltpu
from jax.experimental.pallas import tpu_sc as plsc

HP, WP, HEADS = 32, 32, 16
NB = 32
N = HP * WP
WIDE = 2048


def _sc_gather_body(idxv_hbm, table_hbm, w2_hbm,
                    rowidx_v, ival_v, ival2_v, rows_v, trans_v, sem):
    a = lax.axis_index("s") * 2 + lax.axis_index("c")
    for ch in range(4):
        d = lax.iota(jnp.int32, 16) + 16 * ch
        lo = 32 * ((31 - d) * 32 + a)
        hi = 32 * a + (d - 31)
        rowidx_v[pl.ds(16 * ch, 16)] = jnp.where(d <= 31, lo, hi)
    pltpu.async_copy(idxv_hbm.at[rowidx_v], ival_v, sem).wait()

    for ch in range(16):
        for u in range(8):
            d, k = (ch * 128 + u * 16) // 32, (u * 16) % 32
            ival2_v[ch, pl.ds(u * 16, 16)] = ival_v[d, pl.ds(k, 16)]

    for ch in range(16):
        pltpu.make_async_copy(
            table_hbm.at[ival2_v.at[ch]],
            rows_v.at[pl.ds(ch * 128, 128)], sem).start()

    lanes = lax.iota(jnp.int32, 16)
    for ch in range(16):
        pltpu.make_async_copy(
            table_hbm.at[ival2_v.at[ch]],
            rows_v.at[pl.ds(ch * 128, 128)], sem).wait()

        def trh(h, carry, ch=ch):
            hvec = jnp.zeros((16,), jnp.int32) + h
            for u in range(8):
                c = ch * 8 + u
                x = plsc.load_gather(rows_v, [c * 16 + lanes, hvec])
                trans_v[h, pl.ds(pl.multiple_of(c * 16, 16), 16)] = x
            return carry
        lax.fori_loop(0, HEADS, trh, 0)

    for h in range(HEADS):
        pltpu.make_async_copy(trans_v.at[h], w2_hbm.at[h, a], sem).start()
    for h in range(HEADS):
        pltpu.make_async_copy(trans_v.at[h], w2_hbm.at[h, a], sem).wait()


def _expand_body(w_ref, o_ref):
    y = w_ref[0]
    for k in range(NB):
        off = 32 * (31 - k)
        o_ref[0, k] = y[:, off:off + 1024]


def kernel(bias_table, index):
    idxv = index.reshape((N * N) // 32, 32)
    sc_gather = pl.kernel(
        _sc_gather_body,
        out_type=jax.ShapeDtypeStruct((HEADS, 32, WIDE), jnp.float32),
        mesh=plsc.VectorSubcoreMesh(core_axis_name="c", subcore_axis_name="s"),
        compiler_params=pltpu.CompilerParams(
            needs_layout_passes=False, use_tc_tiling_on_sc=False),
        scratch_types=[
            pltpu.VMEM((64,), jnp.int32),
            pltpu.VMEM((64, 32), jnp.int32),
            pltpu.VMEM((16, 128), jnp.int32),
            pltpu.VMEM((WIDE, HEADS), jnp.float32),
            pltpu.VMEM((HEADS, WIDE), jnp.float32),
            pltpu.SemaphoreType.DMA,
        ],
    )
    w2 = sc_gather(idxv, bias_table)
    out4 = pl.pallas_call(
        _expand_body,
        grid=(HEADS,),
        in_specs=[pl.BlockSpec((1, 32, WIDE), lambda h: (h, 0, 0))],
        out_specs=pl.BlockSpec((1, NB, 32, N), lambda h: (h, 0, 0, 0)),
        out_shape=jax.ShapeDtypeStruct((HEADS, NB, 32, N), jnp.float32),
        compiler_params=pltpu.CompilerParams(
            dimension_semantics=("parallel",)),
    )(w2)
    return out4.reshape(HEADS, N, N)
